# Initial kernel scaffold; baseline (speedup 1.0000x reference)
import jax
import jax.numpy as jnp
from jax.experimental import pallas as pl


def kernel(x, adj, W1, b1, W2, b2, W3, b3, W4, b4):
    raise NotImplementedError("write your pallas kernel here")


# speedup vs baseline: 1.3797x; 1.3797x over previous
import jax
import jax.numpy as jnp
from jax.experimental import pallas as pl

B, N, F = 16, 2048, 128
NUM_CLASSES = 16


def _fused_gcn_kernel(x_ref, adj_ref, w1_ref, b1_ref, w2_ref, b2_ref,
                      w3_ref, b3_ref, w4_ref, b4_ref,
                      dec_ref, act_ref):
    a = adj_ref[0]
    xb = x_ref[0]

    def layer(h, w_ref, b_ref, act):
        t = jnp.dot(h, w_ref[...], preferred_element_type=jnp.float32)
        o = jnp.dot(a, t, preferred_element_type=jnp.float32) + b_ref[...]
        return jnp.maximum(o, 0.0) if act else o

    h = layer(xb, w1_ref, b1_ref, True)
    h = layer(h, w2_ref, b2_ref, True)
    h = layer(h, w3_ref, b3_ref, True)
    dec = layer(h, w4_ref, b4_ref, False)
    dec_ref[0] = dec

    lane = jax.lax.broadcasted_iota(jnp.int32, (N, F), 1)
    is_cls = lane < NUM_CLASSES
    neg = jnp.float32(-1e30)
    m = jnp.max(jnp.where(is_cls, dec, neg), axis=-1, keepdims=True)
    e = jnp.exp(dec - m)
    denom = jnp.sum(jnp.where(is_cls, e, 0.0), axis=-1, keepdims=True)
    act_ref[0] = jnp.where(is_cls, e / denom, jax.nn.sigmoid(dec))


def kernel(x, adj, W1, b1, W2, b2, W3, b3, W4, b4):
    b1r = b1.reshape(1, -1)
    b2r = b2.reshape(1, -1)
    b3r = b3.reshape(1, -1)
    b4r = b4.reshape(1, -1)

    full = lambda s: pl.BlockSpec(s, lambda i: (0,) * len(s))
    grid_spec = pl.GridSpec(
        grid=(B,),
        in_specs=[
            pl.BlockSpec((1, N, F), lambda i: (i, 0, 0)),
            pl.BlockSpec((1, N, N), lambda i: (i, 0, 0)),
            full(W1.shape), full(b1r.shape),
            full(W2.shape), full(b2r.shape),
            full(W3.shape), full(b3r.shape),
            full(W4.shape), full(b4r.shape),
        ],
        out_specs=[
            pl.BlockSpec((1, N, F), lambda i: (i, 0, 0)),
            pl.BlockSpec((1, N, F), lambda i: (i, 0, 0)),
        ],
    )
    out_shape = [
        jax.ShapeDtypeStruct((B, N, F), jnp.float32),
        jax.ShapeDtypeStruct((B, N, F), jnp.float32),
    ]
    dec, act = pl.pallas_call(
        _fused_gcn_kernel,
        grid_spec=grid_spec,
        out_shape=out_shape,
    )(x, adj, W1, b1r, W2, b2r, W3, b3r, W4, b4r)
    return (dec, act)
